# baseline (speedup 1.0000x reference)
import functools

import jax
import jax.numpy as jnp
from jax.experimental import pallas as pl
from jax.experimental.pallas import tpu as pltpu
from jax.experimental.pallas import tpu_sc as plsc

N_HEAD = 16
CHUNK = 64
TOPK = 8
C = 1024
KV = 8192
NUM_CHUNKS = KV // CHUNK
BLK = 512
NBLK = KV // BLK
HS = C // N_HEAD
SCALE = 1.0 / (HS ** 0.5)


def _dotT(a, b):
    return jax.lax.dot_general(a, b, (((1,), (1,)), ((), ())),
                               preferred_element_type=jnp.float32)


def _dot(a, b):
    return jax.lax.dot_general(a, b, (((1,), (0,)), ((), ())),
                               preferred_element_type=jnp.float32)


def _proj_v_body(x_ref, wv_ref, vtok_ref):
    vtok_ref[...] = _dotT(x_ref[...], wv_ref[...])


def _proj_v(x_row, W_v):
    return pl.pallas_call(
        _proj_v_body,
        out_shape=jax.ShapeDtypeStruct((1, C), jnp.float32),
    )(x_row, W_v)


def _sc_v_append(vc2d, vtok):
    info = plsc.get_sparse_core_info()
    nc = info.num_cores
    mesh = plsc.VectorSubcoreMesh(core_axis_name="c", subcore_axis_name="s")
    rows_w = KV // (nc * info.num_subcores)

    @functools.partial(
        pl.kernel,
        mesh=mesh,
        compiler_params=pltpu.CompilerParams(use_tc_tiling_on_sc=True),
        out_type=jax.ShapeDtypeStruct((KV + 1, C), jnp.float32),
        scratch_types=[pltpu.SemaphoreType.DMA],
    )
    def g(vc_hbm, vtok_hbm, vnew_hbm, sem):
        wid = jax.lax.axis_index("s") * nc + jax.lax.axis_index("c")
        base = wid * rows_w
        cp = pltpu.async_copy(vc_hbm.at[pl.ds(base, rows_w)],
                              vnew_hbm.at[pl.ds(base, rows_w)], sem)

        @pl.when(wid == 0)
        def _():
            pltpu.sync_copy(vtok_hbm, vnew_hbm.at[pl.ds(KV, 1)])

        cp.wait()

    return g(vc2d, vtok)


def _k_side_body(x_ref, wr_ref, wk_ref, kc_ref,
                 knew_ref, q_ref, topk_ref, csums):
    i = pl.program_id(0)

    @pl.when(i < NBLK)
    def _():
        blk = kc_ref[...]
        knew_ref[...] = blk
        ck = jnp.sum(blk.reshape(BLK // CHUNK, CHUNK, C), axis=1)
        csums[pl.ds(i * (BLK // CHUNK), BLK // CHUNK), :] = ck

    @pl.when(i == NBLK)
    def _():
        knew_ref[0:1, :] = _dotT(x_ref[...], wk_ref[...])
        q = _dotT(x_ref[...], wr_ref[...])
        q_ref[...] = q
        s = _dotT(q, csums[...])
        lane = jax.lax.broadcasted_iota(jnp.int32, (1, NUM_CHUNKS), 1)
        lane512 = jax.lax.broadcasted_iota(jnp.int32, (1, TOPK * CHUNK), 1)
        rows = lane512 % CHUNK
        for j in range(TOPK):
            m = jnp.max(s)
            idx = jnp.min(jnp.where(s == m, lane, NUM_CHUNKS))
            rows = rows + jnp.where(lane512 // CHUNK == j, idx * CHUNK, 0)
            s = jnp.where(lane == idx, -jnp.inf, s)
        topk_ref[...] = rows


def _k_side(x_row, W_r, W_k, kc2d):
    return pl.pallas_call(
        _k_side_body,
        grid=(NBLK + 1,),
        in_specs=[
            pl.BlockSpec((1, C), lambda i: (0, 0)),
            pl.BlockSpec((C, C), lambda i: (0, 0)),
            pl.BlockSpec((C, C), lambda i: (0, 0)),
            pl.BlockSpec((BLK, C), lambda i: (jnp.minimum(i, NBLK - 1), 0)),
        ],
        out_specs=[
            pl.BlockSpec((BLK, C), lambda i: (i, 0)),
            pl.BlockSpec((1, C), lambda i: (0, 0)),
            pl.BlockSpec((1, TOPK * CHUNK), lambda i: (0, 0)),
        ],
        out_shape=[
            jax.ShapeDtypeStruct((KV + 1, C), jnp.float32),
            jax.ShapeDtypeStruct((1, C), jnp.float32),
            jax.ShapeDtypeStruct((1, TOPK * CHUNK), jnp.int32),
        ],
        scratch_shapes=[pltpu.VMEM((NUM_CHUNKS, C), jnp.float32)],
    )(x_row, W_r, W_k, kc2d)


def _sc_gather(rows512, kc2d, vc2d):
    info = plsc.get_sparse_core_info()
    nc = info.num_cores
    mesh = plsc.VectorSubcoreMesh(core_axis_name="c", subcore_axis_name="s")
    rows_per_w = 16

    @functools.partial(
        pl.kernel,
        mesh=mesh,
        compiler_params=pltpu.CompilerParams(use_tc_tiling_on_sc=True),
        out_type=(jax.ShapeDtypeStruct((TOPK * CHUNK, C), jnp.float32),
                  jax.ShapeDtypeStruct((TOPK * CHUNK, C), jnp.float32)),
        scratch_types=[
            pltpu.VMEM((rows_per_w,), jnp.int32),
            pltpu.VMEM((rows_per_w, C), jnp.float32),
            pltpu.VMEM((rows_per_w, C), jnp.float32),
            pltpu.SemaphoreType.DMA,
            pltpu.SemaphoreType.DMA,
        ],
    )
    def g(idx_hbm, k_hbm, v_hbm, ksel_hbm, vsel_hbm,
          idx_v, kbuf, vbuf, sem_k, sem_v):
        wid = jax.lax.axis_index("s") * nc + jax.lax.axis_index("c")
        base = wid * rows_per_w
        pltpu.sync_copy(idx_hbm.at[pl.ds(base, rows_per_w)], idx_v)
        cpk = pltpu.async_copy(k_hbm.at[idx_v], kbuf, sem_k)
        cpv = pltpu.async_copy(v_hbm.at[idx_v], vbuf, sem_v)
        cpk.wait()
        cpv.wait()
        pltpu.sync_copy(kbuf, ksel_hbm.at[pl.ds(base, rows_per_w)])
        pltpu.sync_copy(vbuf, vsel_hbm.at[pl.ds(base, rows_per_w)])

    return g(rows512, kc2d, vc2d)


def _attn_body(q_ref, ktok_ref, vtok_ref, ksel_ref, vsel_ref, wo_ref, y_ref):
    q = q_ref[...]
    r_ch = jax.lax.broadcasted_iota(jnp.int32, (C, N_HEAD), 0)
    c_h = jax.lax.broadcasted_iota(jnp.int32, (C, N_HEAD), 1)
    emask = (r_ch // HS == c_h).astype(jnp.float32)
    r_h = jax.lax.broadcasted_iota(jnp.int32, (N_HEAD, C), 0)
    c_ch = jax.lax.broadcasted_iota(jnp.int32, (N_HEAD, C), 1)
    emask_t = (c_ch // HS == r_h).astype(jnp.float32)

    T = TOPK * CHUNK
    logits = _dot(ksel_ref[...] * q, emask) * SCALE
    lcur = _dot(ktok_ref[...] * q, emask) * SCALE
    m = jnp.maximum(jnp.max(logits, axis=0, keepdims=True), lcur)
    p = jnp.exp(logits - m)
    pcur = jnp.exp(lcur - m)
    ssum = jnp.sum(p, axis=0, keepdims=True) + pcur
    wfull = _dot(p, emask_t)
    ws = wfull * vsel_ref[...]
    ones = jnp.ones((1, T), jnp.float32)
    ynum = _dot(ones, ws)
    ynum = ynum + _dot(pcur, emask_t) * vtok_ref[...]
    sden = _dot(ssum, emask_t)
    y = ynum / sden
    y_ref[...] = _dotT(y, wo_ref[...])


def _attn(q_row, ktok, vtok, ksel, vsel, W_o):
    return pl.pallas_call(
        _attn_body,
        out_shape=jax.ShapeDtypeStruct((1, C), jnp.float32),
    )(q_row, ktok, vtok, ksel, vsel, W_o)


def kernel(x, k_cache, v_cache, W_r, W_k, W_v, W_o):
    x_row = x.reshape(1, C)
    kc2d = k_cache.reshape(KV, C)
    vc2d = v_cache.reshape(KV, C)
    vtok_p = _proj_v(x_row, W_v)
    vnew = _sc_v_append(vc2d, vtok_p).reshape(1, KV + 1, C)
    knew, q_row, rows512 = _k_side(x_row, W_r, W_k, kc2d)
    knew = knew.reshape(1, KV + 1, C)
    knew = jax.lax.optimization_barrier(knew)
    vnew = jax.lax.optimization_barrier(vnew)
    ktok = jax.lax.slice(knew, (0, KV, 0), (1, KV + 1, C)).reshape(1, C)
    vtok = jax.lax.slice(vnew, (0, KV, 0), (1, KV + 1, C)).reshape(1, C)
    ksel, vsel = _sc_gather(rows512.reshape(TOPK * CHUNK), kc2d, vc2d)
    y_row = _attn(q_row, ktok, vtok, ksel, vsel, W_o)
    return (y_row.reshape(C), knew, vnew)


# speedup vs baseline: 8.7430x; 8.7430x over previous
import functools

import jax
import jax.numpy as jnp
from jax.experimental import pallas as pl
from jax.experimental.pallas import tpu as pltpu
from jax.experimental.pallas import tpu_sc as plsc

N_HEAD = 16
CHUNK = 64
TOPK = 8
C = 1024
KV = 8192
NUM_CHUNKS = KV // CHUNK
BLK = 1024
NBLK = KV // BLK
HS = C // N_HEAD
SCALE = 1.0 / (HS ** 0.5)


def _dotT(a, b):
    return jax.lax.dot_general(a, b, (((1,), (1,)), ((), ())),
                               preferred_element_type=jnp.float32)


def _dot(a, b):
    return jax.lax.dot_general(a, b, (((1,), (0,)), ((), ())),
                               preferred_element_type=jnp.float32)


def _v_side_body(x_ref, wv_ref, vc_ref, vnew_ref):
    i = pl.program_id(0)

    @pl.when(i < NBLK)
    def _():
        vnew_ref[...] = vc_ref[...]

    @pl.when(i == NBLK)
    def _():
        vnew_ref[0:1, :] = _dotT(x_ref[...], wv_ref[...])


def _v_side(x_row, W_v, vc2d):
    return pl.pallas_call(
        _v_side_body,
        grid=(NBLK + 1,),
        in_specs=[
            pl.BlockSpec((1, C), lambda i: (0, 0)),
            pl.BlockSpec((C, C), lambda i: (0, 0)),
            pl.BlockSpec((BLK, C), lambda i: (jnp.minimum(i, NBLK - 1), 0)),
        ],
        out_specs=pl.BlockSpec((BLK, C), lambda i: (i, 0)),
        out_shape=jax.ShapeDtypeStruct((KV + 1, C), jnp.float32),
    )(x_row, W_v, vc2d)


def _k_side_body(x_ref, wr_ref, wk_ref, kc_ref,
                 knew_ref, q_ref, topk_ref, csums):
    i = pl.program_id(0)

    @pl.when(i < NBLK)
    def _():
        blk = kc_ref[...]
        knew_ref[...] = blk
        ck = jnp.sum(blk.reshape(BLK // CHUNK, CHUNK, C), axis=1)
        csums[pl.ds(i * (BLK // CHUNK), BLK // CHUNK), :] = ck

    @pl.when(i == NBLK)
    def _():
        knew_ref[0:1, :] = _dotT(x_ref[...], wk_ref[...])
        q = _dotT(x_ref[...], wr_ref[...])
        q_ref[...] = q
        s = _dotT(q, csums[...])
        lane = jax.lax.broadcasted_iota(jnp.int32, (1, NUM_CHUNKS), 1)
        lane512 = jax.lax.broadcasted_iota(jnp.int32, (1, TOPK * CHUNK), 1)
        rows = lane512 % CHUNK
        for j in range(TOPK):
            m = jnp.max(s)
            idx = jnp.min(jnp.where(s == m, lane, NUM_CHUNKS))
            rows = rows + jnp.where(lane512 // CHUNK == j, idx * CHUNK, 0)
            s = jnp.where(lane == idx, -jnp.inf, s)
        topk_ref[...] = rows


def _k_side(x_row, W_r, W_k, kc2d):
    return pl.pallas_call(
        _k_side_body,
        grid=(NBLK + 1,),
        in_specs=[
            pl.BlockSpec((1, C), lambda i: (0, 0)),
            pl.BlockSpec((C, C), lambda i: (0, 0)),
            pl.BlockSpec((C, C), lambda i: (0, 0)),
            pl.BlockSpec((BLK, C), lambda i: (jnp.minimum(i, NBLK - 1), 0)),
        ],
        out_specs=[
            pl.BlockSpec((BLK, C), lambda i: (i, 0)),
            pl.BlockSpec((1, C), lambda i: (0, 0)),
            pl.BlockSpec((1, TOPK * CHUNK), lambda i: (0, 0)),
        ],
        out_shape=[
            jax.ShapeDtypeStruct((KV + 1, C), jnp.float32),
            jax.ShapeDtypeStruct((1, C), jnp.float32),
            jax.ShapeDtypeStruct((1, TOPK * CHUNK), jnp.int32),
        ],
        scratch_shapes=[pltpu.VMEM((NUM_CHUNKS, C), jnp.float32)],
    )(x_row, W_r, W_k, kc2d)


def _sc_gather(rows512, kc2d, vc2d):
    info = plsc.get_sparse_core_info()
    nc = info.num_cores
    mesh = plsc.VectorSubcoreMesh(core_axis_name="c", subcore_axis_name="s")
    rows_per_w = 16

    @functools.partial(
        pl.kernel,
        mesh=mesh,
        compiler_params=pltpu.CompilerParams(use_tc_tiling_on_sc=True),
        out_type=(jax.ShapeDtypeStruct((TOPK * CHUNK, C), jnp.float32),
                  jax.ShapeDtypeStruct((TOPK * CHUNK, C), jnp.float32)),
        scratch_types=[
            pltpu.VMEM((rows_per_w,), jnp.int32),
            pltpu.VMEM((rows_per_w, C), jnp.float32),
            pltpu.VMEM((rows_per_w, C), jnp.float32),
            pltpu.SemaphoreType.DMA,
            pltpu.SemaphoreType.DMA,
        ],
    )
    def g(idx_hbm, k_hbm, v_hbm, ksel_hbm, vsel_hbm,
          idx_v, kbuf, vbuf, sem_k, sem_v):
        wid = jax.lax.axis_index("s") * nc + jax.lax.axis_index("c")
        base = wid * rows_per_w
        pltpu.sync_copy(idx_hbm.at[pl.ds(base, rows_per_w)], idx_v)
        cpk = pltpu.async_copy(k_hbm.at[idx_v], kbuf, sem_k)
        cpv = pltpu.async_copy(v_hbm.at[idx_v], vbuf, sem_v)
        cpk.wait()
        cpv.wait()
        pltpu.sync_copy(kbuf, ksel_hbm.at[pl.ds(base, rows_per_w)])
        pltpu.sync_copy(vbuf, vsel_hbm.at[pl.ds(base, rows_per_w)])

    return g(rows512, kc2d, vc2d)


def _attn_body(q_ref, ktok_ref, vtok_ref, ksel_ref, vsel_ref, wo_ref, y_ref):
    q = q_ref[...]
    r_ch = jax.lax.broadcasted_iota(jnp.int32, (C, N_HEAD), 0)
    c_h = jax.lax.broadcasted_iota(jnp.int32, (C, N_HEAD), 1)
    emask = (r_ch // HS == c_h).astype(jnp.float32)
    r_h = jax.lax.broadcasted_iota(jnp.int32, (N_HEAD, C), 0)
    c_ch = jax.lax.broadcasted_iota(jnp.int32, (N_HEAD, C), 1)
    emask_t = (c_ch // HS == r_h).astype(jnp.float32)

    T = TOPK * CHUNK
    logits = _dot(ksel_ref[...] * q, emask) * SCALE
    lcur = _dot(ktok_ref[...] * q, emask) * SCALE
    m = jnp.maximum(jnp.max(logits, axis=0, keepdims=True), lcur)
    p = jnp.exp(logits - m)
    pcur = jnp.exp(lcur - m)
    ssum = jnp.sum(p, axis=0, keepdims=True) + pcur
    wfull = _dot(p, emask_t)
    ws = wfull * vsel_ref[...]
    ones = jnp.ones((1, T), jnp.float32)
    ynum = _dot(ones, ws)
    ynum = ynum + _dot(pcur, emask_t) * vtok_ref[...]
    sden = _dot(ssum, emask_t)
    y = ynum / sden
    y_ref[...] = _dotT(y, wo_ref[...])


def _attn(q_row, ktok, vtok, ksel, vsel, W_o):
    return pl.pallas_call(
        _attn_body,
        out_shape=jax.ShapeDtypeStruct((1, C), jnp.float32),
    )(q_row, ktok, vtok, ksel, vsel, W_o)


def kernel(x, k_cache, v_cache, W_r, W_k, W_v, W_o):
    x_row = x.reshape(1, C)
    kc2d = k_cache.reshape(KV, C)
    vc2d = v_cache.reshape(KV, C)
    vnew = _v_side(x_row, W_v, vc2d).reshape(1, KV + 1, C)
    knew, q_row, rows512 = _k_side(x_row, W_r, W_k, kc2d)
    knew = knew.reshape(1, KV + 1, C)
    knew = jax.lax.optimization_barrier(knew)
    vnew = jax.lax.optimization_barrier(vnew)
    ktok = jax.lax.slice(knew, (0, KV, 0), (1, KV + 1, C)).reshape(1, C)
    vtok = jax.lax.slice(vnew, (0, KV, 0), (1, KV + 1, C)).reshape(1, C)
    ksel, vsel = _sc_gather(rows512.reshape(TOPK * CHUNK), kc2d, vc2d)
    y_row = _attn(q_row, ktok, vtok, ksel, vsel, W_o)
    return (y_row.reshape(C), knew, vnew)
